# baseline (speedup 1.0000x reference)
import functools
import jax
import jax.numpy as jnp
from jax.experimental import pallas as pl

B, N, INPUT_DIM = 16, 32768, 32
K_DIM, D_MODEL = 16, 1024
HIDDEN = 64
K_SEL, R_SEL, LAM = 8, 1.0, 0.5
MAX_PROXY = 16

NCHUNK = 32
CH = N // NCHUNK


def _body(x_ref, W1_ref, b1_ref, W2_ref, b2_ref, Wl_ref, bl_ref, Wp_ref,
          bp_ref, y_ref, tok_ref, s_ref):
    for c in range(NCHUNK):
        xc = x_ref[0, pl.ds(c * CH, CH), :]
        h = jnp.tanh(
            jnp.dot(xc, W1_ref[...], preferred_element_type=jnp.float32)
            + b1_ref[...][None, :])
        e = (jnp.dot(h, W2_ref[...], preferred_element_type=jnp.float32)
             + b2_ref[...][None, :])
        s = jnp.maximum(e, 0.0) + jnp.log1p(jnp.exp(-jnp.abs(e)))
        s_ref[:, pl.ds(c, 1)] = s

    sal = s_ref[...]

    t = sal * (R_SEL / LAM)
    m = jnp.max(t)
    p = jnp.exp(t - m)
    z = jnp.sum(p)
    y_ref[0] = p * (K_SEL / z)

    n_flat = (jax.lax.broadcasted_iota(jnp.int32, (CH, NCHUNK), 1) * CH
              + jax.lax.broadcasted_iota(jnp.int32, (CH, NCHUNK), 0))

    work = sal
    neg = jnp.float32(-jnp.inf)
    big = jnp.int32(2 ** 30)
    rows, svals, posvals, cumvals = [], [], [], []
    inv_nm1 = jnp.float32(1.0 / (N - 1))
    for _ in range(MAX_PROXY):
        mx = jnp.max(work)
        idx = jnp.min(jnp.where((work == mx) & (n_flat < big), n_flat, big))
        work = jnp.where(n_flat == idx, neg, work)
        rows.append(x_ref[0, pl.ds(idx, 1), :])
        svals.append(mx)
        posvals.append(idx.astype(jnp.float32) * inv_nm1)
        cumvals.append(jnp.sum(jnp.where(n_flat <= idx, sal, 0.0)))

    xg = jnp.concatenate(rows, axis=0)
    s16 = jnp.stack(svals)[:, None]
    pos16 = jnp.stack(posvals)[:, None]
    cum16 = (jnp.stack(cumvals) * jnp.float32(1.0 / N))[:, None]

    nrm = jnp.sqrt(jnp.sum(xg * xg, axis=1, keepdims=True)
                   + s16 * s16 + pos16 * pos16 + cum16 * cum16)
    inv = 1.0 / (nrm + 1e-6)
    Wl = Wl_ref[...]
    lift_pre = (jnp.dot(xg, Wl[0:INPUT_DIM, :],
                        preferred_element_type=jnp.float32)
                + s16 * Wl[INPUT_DIM:INPUT_DIM + 1, :]
                + pos16 * Wl[INPUT_DIM + 1:INPUT_DIM + 2, :]
                + cum16 * Wl[INPUT_DIM + 2:INPUT_DIM + 3, :])
    lifted = jnp.tanh(inv * lift_pre + bl_ref[...][None, :])
    tok_ref[0] = (jnp.dot(lifted, Wp_ref[...],
                          preferred_element_type=jnp.float32)
                  + bp_ref[...][None, :])


@functools.partial(jax.jit, static_argnames=("interpret",))
def _run(x, W1, b1, W2, b2, W_lift, b_lift, Wp, bp, interpret=False):
    y3, tokens = pl.pallas_call(
        _body,
        grid=(B,),
        in_specs=[
            pl.BlockSpec((1, N, INPUT_DIM), lambda b: (b, 0, 0)),
            pl.BlockSpec((INPUT_DIM, HIDDEN), lambda b: (0, 0)),
            pl.BlockSpec((HIDDEN,), lambda b: (0,)),
            pl.BlockSpec((HIDDEN, 1), lambda b: (0, 0)),
            pl.BlockSpec((1,), lambda b: (0,)),
            pl.BlockSpec((INPUT_DIM + 3, K_DIM), lambda b: (0, 0)),
            pl.BlockSpec((K_DIM,), lambda b: (0,)),
            pl.BlockSpec((K_DIM, D_MODEL), lambda b: (0, 0)),
            pl.BlockSpec((D_MODEL,), lambda b: (0,)),
        ],
        out_specs=[
            pl.BlockSpec((1, CH, NCHUNK), lambda b: (b, 0, 0)),
            pl.BlockSpec((1, MAX_PROXY, D_MODEL), lambda b: (b, 0, 0)),
        ],
        out_shape=[
            jax.ShapeDtypeStruct((B, CH, NCHUNK), jnp.float32),
            jax.ShapeDtypeStruct((B, MAX_PROXY, D_MODEL), jnp.float32),
        ],
        scratch_shapes=[pltpu_vmem((CH, NCHUNK), jnp.float32)],
        interpret=interpret,
    )(x, W1, b1, W2, b2, W_lift, b_lift, Wp, bp)
    y_star = jnp.transpose(y3, (0, 2, 1)).reshape(B, N)
    return tokens, y_star


def pltpu_vmem(shape, dtype):
    from jax.experimental.pallas import tpu as pltpu
    return pltpu.VMEM(shape, dtype)


def kernel(x, W1, b1, W2, b2, W_lift, b_lift, Wp, bp):
    return _run(x, W1, b1, W2, b2, W_lift, b_lift, Wp, bp)


# speedup vs baseline: 1.1557x; 1.1557x over previous
import functools
import jax
import jax.numpy as jnp
from jax.experimental import pallas as pl
from jax.experimental.pallas import tpu as pltpu

B, N, INPUT_DIM = 16, 32768, 32
K_DIM, D_MODEL = 16, 1024
HIDDEN = 64
K_SEL, R_SEL, LAM = 8, 1.0, 0.5
MAX_PROXY = 16

PACK = 4
NP = N // PACK
NCHUNK = 32
PCH = NP // NCHUNK
SROWS, SCOLS = NCHUNK, PACK * PCH


def _body(x_ref, W1_ref, b1_ref, W2t_ref, b2_ref, Wl_ref, bl_ref, Wp_ref,
          bp_ref, y_ref, tok_ref, s_ref):
    dn = (((1,), (1,)), ((), ()))
    for c in range(NCHUNK):
        xc = x_ref[0, pl.ds(c * PCH, PCH), :]
        for g in range(PACK):
            xg = xc[:, g * INPUT_DIM:(g + 1) * INPUT_DIM]
            h = jnp.tanh(
                jnp.dot(xg, W1_ref[...], preferred_element_type=jnp.float32)
                + b1_ref[...][None, :])
            e = jax.lax.dot_general(
                W2t_ref[...], h, dn,
                preferred_element_type=jnp.float32) + b2_ref[0]
            s = jnp.maximum(e, 0.0) + jnp.log1p(jnp.exp(-jnp.abs(e)))
            s_ref[pl.ds(c, 1), pl.ds(g * PCH, PCH)] = s

    sal = s_ref[...]

    t = sal * (R_SEL / LAM)
    m = jnp.max(t)
    p = jnp.exp(t - m)
    z = jnp.sum(p)
    y_ref[0] = p * (K_SEL / z)

    i0 = jax.lax.broadcasted_iota(jnp.int32, (SROWS, SCOLS), 0)
    i1 = jax.lax.broadcasted_iota(jnp.int32, (SROWS, SCOLS), 1)
    n_flat = i0 * (PACK * PCH) + (i1 % PCH) * PACK + i1 // PCH

    work = sal
    neg = jnp.float32(-jnp.inf)
    big = jnp.int32(2 ** 30)
    lane_i = jax.lax.broadcasted_iota(jnp.int32, (128, INPUT_DIM), 0)
    feat_i = jax.lax.broadcasted_iota(jnp.int32, (128, INPUT_DIM), 1)
    rows, svals, posvals, cumvals = [], [], [], []
    inv_nm1 = jnp.float32(1.0 / (N - 1))
    for _ in range(MAX_PROXY):
        mx = jnp.max(work)
        idx = jnp.min(jnp.where(work == mx, n_flat, big))
        work = jnp.where(n_flat == idx, neg, work)
        row128 = x_ref[0, pl.ds(idx // PACK, 1), :]
        sel = (lane_i == (idx % PACK) * INPUT_DIM + feat_i)
        rows.append(jnp.dot(row128, sel.astype(jnp.float32),
                            preferred_element_type=jnp.float32))
        svals.append(mx)
        posvals.append(idx.astype(jnp.float32) * inv_nm1)
        cumvals.append(jnp.sum(jnp.where(n_flat <= idx, sal, 0.0)))

    xg16 = jnp.concatenate(rows, axis=0)
    s16 = jnp.stack(svals)[:, None]
    pos16 = jnp.stack(posvals)[:, None]
    cum16 = (jnp.stack(cumvals) * jnp.float32(1.0 / N))[:, None]

    nrm = jnp.sqrt(jnp.sum(xg16 * xg16, axis=1, keepdims=True)
                   + s16 * s16 + pos16 * pos16 + cum16 * cum16)
    inv = 1.0 / (nrm + 1e-6)
    Wl = Wl_ref[...]
    lift_pre = (jnp.dot(xg16, Wl[0:INPUT_DIM, :],
                        preferred_element_type=jnp.float32)
                + s16 * Wl[INPUT_DIM:INPUT_DIM + 1, :]
                + pos16 * Wl[INPUT_DIM + 1:INPUT_DIM + 2, :]
                + cum16 * Wl[INPUT_DIM + 2:INPUT_DIM + 3, :])
    lifted = jnp.tanh(inv * lift_pre + bl_ref[...][None, :])
    tok_ref[0] = (jnp.dot(lifted, Wp_ref[...],
                          preferred_element_type=jnp.float32)
                  + bp_ref[...][None, :])


@functools.partial(jax.jit, static_argnames=("interpret",))
def _run(x, W1, b1, W2, b2, W_lift, b_lift, Wp, bp, interpret=False):
    x_p = x.reshape(B, NP, PACK * INPUT_DIM)
    W2t = W2.T
    y3, tokens = pl.pallas_call(
        _body,
        grid=(B,),
        in_specs=[
            pl.BlockSpec((1, NP, PACK * INPUT_DIM), lambda b: (b, 0, 0)),
            pl.BlockSpec((INPUT_DIM, HIDDEN), lambda b: (0, 0)),
            pl.BlockSpec((HIDDEN,), lambda b: (0,)),
            pl.BlockSpec((1, HIDDEN), lambda b: (0, 0)),
            pl.BlockSpec((1,), lambda b: (0,)),
            pl.BlockSpec((INPUT_DIM + 3, K_DIM), lambda b: (0, 0)),
            pl.BlockSpec((K_DIM,), lambda b: (0,)),
            pl.BlockSpec((K_DIM, D_MODEL), lambda b: (0, 0)),
            pl.BlockSpec((D_MODEL,), lambda b: (0,)),
        ],
        out_specs=[
            pl.BlockSpec((1, SROWS, SCOLS), lambda b: (b, 0, 0)),
            pl.BlockSpec((1, MAX_PROXY, D_MODEL), lambda b: (b, 0, 0)),
        ],
        out_shape=[
            jax.ShapeDtypeStruct((B, SROWS, SCOLS), jnp.float32),
            jax.ShapeDtypeStruct((B, MAX_PROXY, D_MODEL), jnp.float32),
        ],
        scratch_shapes=[pltpu.VMEM((SROWS, SCOLS), jnp.float32)],
        interpret=interpret,
    )(x_p, W1, b1, W2t, b2, W_lift, b_lift, Wp, bp)
    y_star = (y3.reshape(B, SROWS, PACK, PCH)
              .transpose(0, 1, 3, 2)
              .reshape(B, N))
    return tokens, y_star


def kernel(x, W1, b1, W2, b2, W_lift, b_lift, Wp, bp):
    return _run(x, W1, b1, W2, b2, W_lift, b_lift, Wp, bp)


# speedup vs baseline: 1.3318x; 1.1523x over previous
import functools
import jax
import jax.numpy as jnp
from jax.experimental import pallas as pl
from jax.experimental.pallas import tpu as pltpu

B, N, INPUT_DIM = 16, 32768, 32
K_DIM, D_MODEL = 16, 1024
HIDDEN = 64
K_SEL, R_SEL, LAM = 8, 1.0, 0.5
MAX_PROXY = 16

PACK = 4
NP = N // PACK
NCHUNK = 8
PCH = NP // NCHUNK
SROWS = PACK * NCHUNK


def _score_body(x_ref, W1b_ref, b1b_ref, W2tb_ref, b2_ref, y_ref, spc_ref,
                idx_ref, s_ref):
    dn = (((1,), (1,)), ((), ()))
    for c in range(NCHUNK):
        xc = x_ref[0, pl.ds(c * PCH, PCH), :]
        h = jnp.tanh(
            jnp.dot(xc, W1b_ref[...], preferred_element_type=jnp.float32)
            + b1b_ref[...][None, :])
        e = jax.lax.dot_general(
            W2tb_ref[...], h, dn,
            preferred_element_type=jnp.float32) + b2_ref[0]
        s = jnp.maximum(e, 0.0) + jnp.log1p(jnp.exp(-jnp.abs(e)))
        s_ref[pl.ds(PACK * c, PACK), :] = s

    sal = s_ref[...]

    t = sal * (R_SEL / LAM)
    m = jnp.max(t, axis=1, keepdims=True).max(axis=0, keepdims=True)
    p = jnp.exp(t - m)
    z = jnp.sum(p, axis=1, keepdims=True).sum(axis=0, keepdims=True)
    y_ref[0] = p * (K_SEL / z)

    i0 = jax.lax.broadcasted_iota(jnp.int32, (SROWS, PCH), 0)
    i1 = jax.lax.broadcasted_iota(jnp.int32, (SROWS, PCH), 1)
    n_flat = (i0 // PACK) * (PACK * PCH) + i1 * PACK + (i0 % PACK)

    work = sal
    neg = jnp.float32(-jnp.inf)
    big = jnp.int32(2 ** 30)
    lane16 = jax.lax.broadcasted_iota(jnp.int32, (1, MAX_PROXY), 1)
    sal_acc = jnp.zeros((1, MAX_PROXY), jnp.float32)
    pos_acc = jnp.zeros((1, MAX_PROXY), jnp.float32)
    cum_acc = jnp.zeros((1, MAX_PROXY), jnp.float32)
    idx_acc = jnp.zeros((1, MAX_PROXY), jnp.int32)
    inv_nm1 = jnp.float32(1.0 / (N - 1))
    for k in range(MAX_PROXY):
        mx = jnp.max(work, axis=1, keepdims=True).max(axis=0, keepdims=True)
        idx = jnp.min(jnp.where(work == mx, n_flat, big),
                      axis=1, keepdims=True).min(axis=0, keepdims=True)
        work = jnp.where(n_flat == idx, neg, work)
        cum = jnp.sum(jnp.where(n_flat <= idx, sal, 0.0),
                      axis=1, keepdims=True).sum(axis=0, keepdims=True)
        hit = lane16 == k
        sal_acc = jnp.where(hit, mx, sal_acc)
        pos_acc = jnp.where(hit, idx.astype(jnp.float32) * inv_nm1, pos_acc)
        cum_acc = jnp.where(hit, cum * jnp.float32(1.0 / N), cum_acc)
        idx_acc = jnp.where(hit, idx, idx_acc)

    spc_ref[0] = jnp.concatenate([sal_acc, pos_acc, cum_acc], axis=0)
    idx_ref[0] = idx_acc


def _proj_body(idx_sref, *refs):
    rows = refs[:MAX_PROXY]
    spc_ref, Wl_ref, bl_ref, Wp_ref, bp_ref, tok_ref = refs[MAX_PROXY:]
    xg16 = jnp.concatenate([r[0, 0] for r in rows], axis=0)
    spc = spc_ref[0]
    s16 = jnp.reshape(spc[0:1, :], (MAX_PROXY, 1))
    pos16 = jnp.reshape(spc[1:2, :], (MAX_PROXY, 1))
    cum16 = jnp.reshape(spc[2:3, :], (MAX_PROXY, 1))
    nrm = jnp.sqrt(jnp.sum(xg16 * xg16, axis=1, keepdims=True)
                   + s16 * s16 + pos16 * pos16 + cum16 * cum16)
    inv = 1.0 / (nrm + 1e-6)
    Wl = Wl_ref[...]
    lift_pre = (jnp.dot(xg16, Wl[0:INPUT_DIM, :],
                        preferred_element_type=jnp.float32)
                + s16 * Wl[INPUT_DIM:INPUT_DIM + 1, :]
                + pos16 * Wl[INPUT_DIM + 1:INPUT_DIM + 2, :]
                + cum16 * Wl[INPUT_DIM + 2:INPUT_DIM + 3, :])
    lifted = jnp.tanh(inv * lift_pre + bl_ref[...][None, :])
    tok_ref[0] = (jnp.dot(lifted, Wp_ref[...],
                          preferred_element_type=jnp.float32)
                  + bp_ref[...][None, :])


@functools.partial(jax.jit, static_argnames=("interpret",))
def _run(x, W1, b1, W2, b2, W_lift, b_lift, Wp, bp, interpret=False):
    x_p = x.reshape(B, NP, PACK * INPUT_DIM)
    zW1 = jnp.zeros((INPUT_DIM, HIDDEN), jnp.float32)
    W1b = jnp.concatenate(
        [jnp.concatenate([W1 if i == j else zW1 for j in range(PACK)], axis=1)
         for i in range(PACK)], axis=0)
    b1b = jnp.tile(b1, PACK)
    zW2 = jnp.zeros((1, HIDDEN), jnp.float32)
    W2tb = jnp.concatenate(
        [jnp.concatenate([W2.T if i == j else zW2 for j in range(PACK)],
                         axis=1) for i in range(PACK)], axis=0)

    y3, spc, idx16 = pl.pallas_call(
        _score_body,
        grid=(B,),
        in_specs=[
            pl.BlockSpec((1, NP, PACK * INPUT_DIM), lambda b: (b, 0, 0)),
            pl.BlockSpec((PACK * INPUT_DIM, PACK * HIDDEN), lambda b: (0, 0)),
            pl.BlockSpec((PACK * HIDDEN,), lambda b: (0,)),
            pl.BlockSpec((PACK, PACK * HIDDEN), lambda b: (0, 0)),
            pl.BlockSpec((1,), lambda b: (0,)),
        ],
        out_specs=[
            pl.BlockSpec((1, SROWS, PCH), lambda b: (b, 0, 0)),
            pl.BlockSpec((1, 3, MAX_PROXY), lambda b: (b, 0, 0)),
            pl.BlockSpec((1, 1, MAX_PROXY), lambda b: (b, 0, 0)),
        ],
        out_shape=[
            jax.ShapeDtypeStruct((B, SROWS, PCH), jnp.float32),
            jax.ShapeDtypeStruct((B, 3, MAX_PROXY), jnp.float32),
            jax.ShapeDtypeStruct((B, 1, MAX_PROXY), jnp.int32),
        ],
        scratch_shapes=[pltpu.VMEM((SROWS, PCH), jnp.float32)],
        interpret=interpret,
    )(x_p, W1b, b1b, W2tb, b2)

    y_star = (y3.reshape(B, NCHUNK, PACK, PCH)
              .transpose(0, 1, 3, 2)
              .reshape(B, N))

    x4 = x.reshape(B, N, 1, INPUT_DIM)

    def row_spec(k):
        return pl.BlockSpec((1, 1, 1, INPUT_DIM),
                            lambda b, idx: (b, idx[b, 0, k], 0, 0))

    tokens = pl.pallas_call(
        _proj_body,
        grid_spec=pltpu.PrefetchScalarGridSpec(
            num_scalar_prefetch=1,
            grid=(B,),
            in_specs=[row_spec(k) for k in range(MAX_PROXY)] + [
                pl.BlockSpec((1, 3, MAX_PROXY), lambda b, idx: (b, 0, 0)),
                pl.BlockSpec((INPUT_DIM + 3, K_DIM), lambda b, idx: (0, 0)),
                pl.BlockSpec((K_DIM,), lambda b, idx: (0,)),
                pl.BlockSpec((K_DIM, D_MODEL), lambda b, idx: (0, 0)),
                pl.BlockSpec((D_MODEL,), lambda b, idx: (0,)),
            ],
            out_specs=pl.BlockSpec((1, MAX_PROXY, D_MODEL),
                                   lambda b, idx: (b, 0, 0)),
        ),
        out_shape=jax.ShapeDtypeStruct((B, MAX_PROXY, D_MODEL), jnp.float32),
        interpret=interpret,
    )(idx16, *([x4] * MAX_PROXY), spc, W_lift, b_lift, Wp, bp)

    return tokens, y_star


def kernel(x, W1, b1, W2, b2, W_lift, b_lift, Wp, bp):
    return _run(x, W1, b1, W2, b2, W_lift, b_lift, Wp, bp)


# speedup vs baseline: 1.5272x; 1.1468x over previous
import functools
import jax
import jax.numpy as jnp
from jax.experimental import pallas as pl
from jax.experimental.pallas import tpu as pltpu

B, N, INPUT_DIM = 16, 32768, 32
K_DIM, D_MODEL = 16, 1024
HIDDEN = 64
K_SEL, R_SEL, LAM = 8, 1.0, 0.5
MAX_PROXY = 16

NCHUNK = 16
CH = N // NCHUNK
SROWS = N // 128


def _score_body(x_ref, W1t_ref, b1_ref, W2t_ref, b2_ref, y_ref, spc_ref,
                idx_ref, s_ref):
    for c in range(NCHUNK):
        xc = x_ref[0, pl.ds(c * CH, CH), :]
        ht = jnp.tanh(jax.lax.dot_general(
            W1t_ref[...], xc, (((1,), (1,)), ((), ())),
            preferred_element_type=jnp.float32)
            + b1_ref[...])
        e = jax.lax.dot_general(
            W2t_ref[...], ht, (((1,), (0,)), ((), ())),
            preferred_element_type=jnp.float32) + b2_ref[0]
        s = jnp.maximum(e, 0.0) + jnp.log1p(jnp.exp(-jnp.abs(e)))
        s_ref[pl.ds((CH // 128) * c, CH // 128), :] = s.reshape(CH // 128,
                                                               128)

    sal = s_ref[...]

    t = sal * (R_SEL / LAM)
    m = jnp.max(t, axis=1, keepdims=True).max(axis=0, keepdims=True)
    p = jnp.exp(t - m)
    z = jnp.sum(p, axis=1, keepdims=True).sum(axis=0, keepdims=True)
    y_ref[0] = p * (K_SEL / z)

    i0 = jax.lax.broadcasted_iota(jnp.int32, (SROWS, 128), 0)
    i1 = jax.lax.broadcasted_iota(jnp.int32, (SROWS, 128), 1)
    n_flat = i0 * 128 + i1

    work = sal
    neg = jnp.float32(-jnp.inf)
    big = jnp.int32(2 ** 30)
    lane16 = jax.lax.broadcasted_iota(jnp.int32, (1, MAX_PROXY), 1)
    del i0, i1
    sal_acc = jnp.zeros((1, MAX_PROXY), jnp.float32)
    pos_acc = jnp.zeros((1, MAX_PROXY), jnp.float32)
    cum_acc = jnp.zeros((1, MAX_PROXY), jnp.float32)
    idx_acc = jnp.zeros((1, MAX_PROXY), jnp.int32)
    inv_nm1 = jnp.float32(1.0 / (N - 1))
    for k in range(MAX_PROXY):
        mx = jnp.max(work, axis=1, keepdims=True).max(axis=0, keepdims=True)
        idx = jnp.min(jnp.where(work == mx, n_flat, big),
                      axis=1, keepdims=True).min(axis=0, keepdims=True)
        work = jnp.where(n_flat == idx, neg, work)
        cum = jnp.sum(jnp.where(n_flat <= idx, sal, 0.0),
                      axis=1, keepdims=True).sum(axis=0, keepdims=True)
        hit = lane16 == k
        sal_acc = jnp.where(hit, mx, sal_acc)
        pos_acc = jnp.where(hit, idx.astype(jnp.float32) * inv_nm1, pos_acc)
        cum_acc = jnp.where(hit, cum * jnp.float32(1.0 / N), cum_acc)
        idx_acc = jnp.where(hit, idx, idx_acc)

    spc_ref[0] = jnp.concatenate([sal_acc, pos_acc, cum_acc], axis=0)
    idx_ref[0] = idx_acc


def _proj_body(idx_sref, *refs):
    rows = refs[:MAX_PROXY]
    spc_ref, Wl_ref, bl_ref, Wp_ref, bp_ref, tok_ref = refs[MAX_PROXY:]
    xg16 = jnp.concatenate([r[0, 0] for r in rows], axis=0)
    spc = spc_ref[0]
    s16 = jnp.reshape(spc[0:1, :], (MAX_PROXY, 1))
    pos16 = jnp.reshape(spc[1:2, :], (MAX_PROXY, 1))
    cum16 = jnp.reshape(spc[2:3, :], (MAX_PROXY, 1))
    nrm = jnp.sqrt(jnp.sum(xg16 * xg16, axis=1, keepdims=True)
                   + s16 * s16 + pos16 * pos16 + cum16 * cum16)
    inv = 1.0 / (nrm + 1e-6)
    Wl = Wl_ref[...]
    lift_pre = (jnp.dot(xg16, Wl[0:INPUT_DIM, :],
                        preferred_element_type=jnp.float32)
                + s16 * Wl[INPUT_DIM:INPUT_DIM + 1, :]
                + pos16 * Wl[INPUT_DIM + 1:INPUT_DIM + 2, :]
                + cum16 * Wl[INPUT_DIM + 2:INPUT_DIM + 3, :])
    lifted = jnp.tanh(inv * lift_pre + bl_ref[...][None, :])
    tok_ref[0] = (jnp.dot(lifted, Wp_ref[...],
                          preferred_element_type=jnp.float32)
                  + bp_ref[...][None, :])


@functools.partial(jax.jit, static_argnames=("interpret",))
def _run(x, W1, b1, W2, b2, W_lift, b_lift, Wp, bp, interpret=False):
    y3, spc, idx16 = pl.pallas_call(
        _score_body,
        grid=(B,),
        in_specs=[
            pl.BlockSpec((1, N, INPUT_DIM), lambda b: (b, 0, 0)),
            pl.BlockSpec((HIDDEN, INPUT_DIM), lambda b: (0, 0)),
            pl.BlockSpec((HIDDEN, 1), lambda b: (0, 0)),
            pl.BlockSpec((1, HIDDEN), lambda b: (0, 0)),
            pl.BlockSpec((1,), lambda b: (0,)),
        ],
        out_specs=[
            pl.BlockSpec((1, SROWS, 128), lambda b: (b, 0, 0)),
            pl.BlockSpec((1, 3, MAX_PROXY), lambda b: (b, 0, 0)),
            pl.BlockSpec((1, 1, MAX_PROXY), lambda b: (b, 0, 0)),
        ],
        out_shape=[
            jax.ShapeDtypeStruct((B, SROWS, 128), jnp.float32),
            jax.ShapeDtypeStruct((B, 3, MAX_PROXY), jnp.float32),
            jax.ShapeDtypeStruct((B, 1, MAX_PROXY), jnp.int32),
        ],
        scratch_shapes=[pltpu.VMEM((SROWS, 128), jnp.float32)],
        interpret=interpret,
    )(x, W1.T, b1[:, None], W2.T, b2)

    y_star = y3.reshape(B, N)

    x4 = x.reshape(B, N, 1, INPUT_DIM)

    def row_spec(k):
        return pl.BlockSpec((1, 1, 1, INPUT_DIM),
                            lambda b, idx: (b, idx[b, 0, k], 0, 0))

    tokens = pl.pallas_call(
        _proj_body,
        grid_spec=pltpu.PrefetchScalarGridSpec(
            num_scalar_prefetch=1,
            grid=(B,),
            in_specs=[row_spec(k) for k in range(MAX_PROXY)] + [
                pl.BlockSpec((1, 3, MAX_PROXY), lambda b, idx: (b, 0, 0)),
                pl.BlockSpec((INPUT_DIM + 3, K_DIM), lambda b, idx: (0, 0)),
                pl.BlockSpec((K_DIM,), lambda b, idx: (0,)),
                pl.BlockSpec((K_DIM, D_MODEL), lambda b, idx: (0, 0)),
                pl.BlockSpec((D_MODEL,), lambda b, idx: (0,)),
            ],
            out_specs=pl.BlockSpec((1, MAX_PROXY, D_MODEL),
                                   lambda b, idx: (b, 0, 0)),
        ),
        out_shape=jax.ShapeDtypeStruct((B, MAX_PROXY, D_MODEL), jnp.float32),
        interpret=interpret,
    )(idx16, *([x4] * MAX_PROXY), spc, W_lift, b_lift, Wp, bp)

    return tokens, y_star


def kernel(x, W1, b1, W2, b2, W_lift, b_lift, Wp, bp):
    return _run(x, W1, b1, W2, b2, W_lift, b_lift, Wp, bp)
